# Initial kernel scaffold; baseline (speedup 1.0000x reference)
import jax
import jax.numpy as jnp
from jax.experimental import pallas as pl


def kernel(pooled_map, sparse_pattern, original_height, original_width):
    raise NotImplementedError("write your pallas kernel here")


# speedup vs baseline: 3.1953x; 3.1953x over previous
import functools

import jax
import jax.numpy as jnp
from jax.experimental import pallas as pl
from jax.experimental.pallas import tpu as pltpu

SPACING = 2


def _unpool_body(ph, pw, oh, ow, lims_ref, pooled_ref, sp_ref, out_ref):
    h_lim = lims_ref[0]
    w_lim = lims_ref[1]
    m = (pooled_ref[0] > 0.5).astype(jnp.float32)

    i = jax.lax.broadcasted_iota(jnp.int32, (oh, ph), 0)
    j = jax.lax.broadcasted_iota(jnp.int32, (oh, ph), 1)
    eh = ((j == i // SPACING) & (i < h_lim)).astype(jnp.float32)
    t = jnp.dot(eh, m, preferred_element_type=jnp.float32)

    jw = jax.lax.broadcasted_iota(jnp.int32, (pw, ow), 0)
    kw = jax.lax.broadcasted_iota(jnp.int32, (pw, ow), 1)
    ew = ((jw == kw // SPACING) & (kw < w_lim)).astype(jnp.float32)
    up = jnp.dot(t, ew, preferred_element_type=jnp.float32)

    out_ref[0] = jnp.where(up > 0.5, sp_ref[0], 0.0)


@jax.jit
def _unpool(pooled_map, sparse_pattern, original_height, original_width):
    b, c, ph, pw = pooled_map.shape
    oh, ow = sparse_pattern.shape[2], sparse_pattern.shape[3]
    n = b * c
    pooled3 = pooled_map.reshape(n, ph, pw)
    sp3 = sparse_pattern.reshape(n, oh, ow)
    lims = jnp.stack([
        jnp.asarray(original_height, jnp.int32),
        jnp.asarray(original_width, jnp.int32),
    ])

    body = functools.partial(_unpool_body, ph, pw, oh, ow)
    out = pl.pallas_call(
        body,
        grid=(n,),
        in_specs=[
            pl.BlockSpec(memory_space=pltpu.SMEM),
            pl.BlockSpec((1, ph, pw), lambda i: (i, 0, 0)),
            pl.BlockSpec((1, oh, ow), lambda i: (i, 0, 0)),
        ],
        out_specs=pl.BlockSpec((1, oh, ow), lambda i: (i, 0, 0)),
        out_shape=jax.ShapeDtypeStruct((n, oh, ow), pooled_map.dtype),
    )(lims, pooled3, sp3)
    return out.reshape(b, c, oh, ow)


def kernel(pooled_map, sparse_pattern, original_height, original_width):
    return _unpool(pooled_map, sparse_pattern, original_height, original_width)


# speedup vs baseline: 10.1766x; 3.1849x over previous
import functools

import jax
import jax.numpy as jnp
from jax.experimental import pallas as pl

SPACING = 2
IMGS_PER_STEP = 8


def _unpool_body(g, eh_ref, ew_ref, pooled_ref, sp_ref, out_ref):
    eh = eh_ref[...]
    ew = ew_ref[...]
    for k in range(g):
        m = (pooled_ref[k] > 0.5).astype(jnp.float32)
        t = jnp.dot(eh, m, preferred_element_type=jnp.float32)
        up = jnp.dot(t, ew, preferred_element_type=jnp.float32)
        out_ref[k] = sp_ref[k] * up


@jax.jit
def _unpool(pooled_map, sparse_pattern, original_height, original_width):
    b, c, ph, pw = pooled_map.shape
    oh, ow = sparse_pattern.shape[2], sparse_pattern.shape[3]
    n = b * c
    pooled3 = pooled_map.reshape(n, ph, pw)
    sp3 = sparse_pattern.reshape(n, oh, ow)

    h_lim = jnp.asarray(original_height, jnp.int32)
    w_lim = jnp.asarray(original_width, jnp.int32)
    i = jax.lax.broadcasted_iota(jnp.int32, (oh, ph), 0)
    j = jax.lax.broadcasted_iota(jnp.int32, (oh, ph), 1)
    eh = ((j == i // SPACING) & (i < h_lim)).astype(jnp.float32)
    jw = jax.lax.broadcasted_iota(jnp.int32, (pw, ow), 0)
    kw = jax.lax.broadcasted_iota(jnp.int32, (pw, ow), 1)
    ew = ((jw == kw // SPACING) & (kw < w_lim)).astype(jnp.float32)

    g = IMGS_PER_STEP
    assert n % g == 0
    body = functools.partial(_unpool_body, g)
    out = pl.pallas_call(
        body,
        grid=(n // g,),
        in_specs=[
            pl.BlockSpec((oh, ph), lambda i: (0, 0)),
            pl.BlockSpec((pw, ow), lambda i: (0, 0)),
            pl.BlockSpec((g, ph, pw), lambda i: (i, 0, 0)),
            pl.BlockSpec((g, oh, ow), lambda i: (i, 0, 0)),
        ],
        out_specs=pl.BlockSpec((g, oh, ow), lambda i: (i, 0, 0)),
        out_shape=jax.ShapeDtypeStruct((n, oh, ow), pooled_map.dtype),
    )(eh, ew, pooled3, sp3)
    return out.reshape(b, c, oh, ow)


def kernel(pooled_map, sparse_pattern, original_height, original_width):
    return _unpool(pooled_map, sparse_pattern, original_height, original_width)


# speedup vs baseline: 12.0235x; 1.1815x over previous
import functools

import jax
import jax.numpy as jnp
from jax.experimental import pallas as pl

SPACING = 2
IMGS_PER_STEP = 16


def _unpool_body(g, eh_ref, ew_ref, pooled_ref, sp_ref, out_ref):
    eh = eh_ref[...]
    ew = ew_ref[...]
    for k in range(g):
        m = (pooled_ref[k] > 0.5).astype(jnp.float32)
        t = jnp.dot(eh, m, preferred_element_type=jnp.float32)
        up = jnp.dot(t, ew, preferred_element_type=jnp.float32)
        out_ref[k] = sp_ref[k] * up


@jax.jit
def _unpool(pooled_map, sparse_pattern, original_height, original_width):
    b, c, ph, pw = pooled_map.shape
    oh, ow = sparse_pattern.shape[2], sparse_pattern.shape[3]
    n = b * c
    pooled3 = pooled_map.reshape(n, ph, pw)
    sp3 = sparse_pattern.reshape(n, oh, ow)

    h_lim = jnp.asarray(original_height, jnp.int32)
    w_lim = jnp.asarray(original_width, jnp.int32)
    i = jax.lax.broadcasted_iota(jnp.int32, (oh, ph), 0)
    j = jax.lax.broadcasted_iota(jnp.int32, (oh, ph), 1)
    eh = ((j == i // SPACING) & (i < h_lim)).astype(jnp.float32)
    jw = jax.lax.broadcasted_iota(jnp.int32, (pw, ow), 0)
    kw = jax.lax.broadcasted_iota(jnp.int32, (pw, ow), 1)
    ew = ((jw == kw // SPACING) & (kw < w_lim)).astype(jnp.float32)

    g = IMGS_PER_STEP
    assert n % g == 0
    body = functools.partial(_unpool_body, g)
    out = pl.pallas_call(
        body,
        grid=(n // g,),
        in_specs=[
            pl.BlockSpec((oh, ph), lambda i: (0, 0)),
            pl.BlockSpec((pw, ow), lambda i: (0, 0)),
            pl.BlockSpec((g, ph, pw), lambda i: (i, 0, 0)),
            pl.BlockSpec((g, oh, ow), lambda i: (i, 0, 0)),
        ],
        out_specs=pl.BlockSpec((g, oh, ow), lambda i: (i, 0, 0)),
        out_shape=jax.ShapeDtypeStruct((n, oh, ow), pooled_map.dtype),
    )(eh, ew, pooled3, sp3)
    return out.reshape(b, c, oh, ow)


def kernel(pooled_map, sparse_pattern, original_height, original_width):
    return _unpool(pooled_map, sparse_pattern, original_height, original_width)


# speedup vs baseline: 12.6544x; 1.0525x over previous
import functools

import jax
import jax.numpy as jnp
from jax.experimental import pallas as pl

SPACING = 2
IMGS_PER_STEP = 32


def _unpool_body(g, eh_ref, ew_ref, pooled_ref, sp_ref, out_ref):
    eh = eh_ref[...]
    ew = ew_ref[...]
    for k in range(g):
        m = (pooled_ref[k] > 0.5).astype(jnp.float32)
        t = jnp.dot(eh, m, preferred_element_type=jnp.float32)
        up = jnp.dot(t, ew, preferred_element_type=jnp.float32)
        out_ref[k] = sp_ref[k] * up


@jax.jit
def _unpool(pooled_map, sparse_pattern, original_height, original_width):
    b, c, ph, pw = pooled_map.shape
    oh, ow = sparse_pattern.shape[2], sparse_pattern.shape[3]
    n = b * c
    pooled3 = pooled_map.reshape(n, ph, pw)
    sp3 = sparse_pattern.reshape(n, oh, ow)

    h_lim = jnp.asarray(original_height, jnp.int32)
    w_lim = jnp.asarray(original_width, jnp.int32)
    i = jax.lax.broadcasted_iota(jnp.int32, (oh, ph), 0)
    j = jax.lax.broadcasted_iota(jnp.int32, (oh, ph), 1)
    eh = ((j == i // SPACING) & (i < h_lim)).astype(jnp.float32)
    jw = jax.lax.broadcasted_iota(jnp.int32, (pw, ow), 0)
    kw = jax.lax.broadcasted_iota(jnp.int32, (pw, ow), 1)
    ew = ((jw == kw // SPACING) & (kw < w_lim)).astype(jnp.float32)

    g = IMGS_PER_STEP
    assert n % g == 0
    body = functools.partial(_unpool_body, g)
    out = pl.pallas_call(
        body,
        grid=(n // g,),
        in_specs=[
            pl.BlockSpec((oh, ph), lambda i: (0, 0)),
            pl.BlockSpec((pw, ow), lambda i: (0, 0)),
            pl.BlockSpec((g, ph, pw), lambda i: (i, 0, 0)),
            pl.BlockSpec((g, oh, ow), lambda i: (i, 0, 0)),
        ],
        out_specs=pl.BlockSpec((g, oh, ow), lambda i: (i, 0, 0)),
        out_shape=jax.ShapeDtypeStruct((n, oh, ow), pooled_map.dtype),
    )(eh, ew, pooled3, sp3)
    return out.reshape(b, c, oh, ow)


def kernel(pooled_map, sparse_pattern, original_height, original_width):
    return _unpool(pooled_map, sparse_pattern, original_height, original_width)
